# baseline (speedup 1.0000x reference)
import functools

import jax
import jax.numpy as jnp
from jax import lax
from jax.experimental import pallas as pl
from jax.experimental.pallas import tpu as pltpu
from jax.experimental.pallas import tpu_sc as plsc

F = 128
FPL = 2048
NK = 11
NL = 3
MAXD = 10.0

NC = 2
NS = 16
NW = NC * NS
CH = 64
DEPTH = 4
DW = 128

BN = 400


NB = 8


def _segsum_kernel(np_rows, rpt, epw, nsc):
    mesh = plsc.VectorSubcoreMesh(core_axis_name="c", subcore_axis_name="s")

    @functools.partial(
        pl.kernel,
        out_type=jax.ShapeDtypeStruct((NC * np_rows, F), jnp.float32),
        mesh=mesh,
        scratch_types=[
            pltpu.VMEM((2 * NB, CH), jnp.int32),
            pltpu.VMEM((2 * NB, CH), jnp.int32),
            [pltpu.VMEM((CH, F), jnp.float32)] * DEPTH,
            pltpu.VMEM_SHARED((np_rows, F), jnp.float32),
            pltpu.SemaphoreType.DMA,
            pltpu.SemaphoreType.DMA,
            [pltpu.SemaphoreType.DMA] * DEPTH,
            [pltpu.SemaphoreType.DMA] * DEPTH,
        ],
    )
    def segsum(x_hbm, src2_hbm, dst2_hbm, z_hbm, out_hbm,
               srcb_v, dstb_v, rows, h_sh, si0, si1, sem_r, sem_s):
        c = lax.axis_index("c")
        s = lax.axis_index("s")
        wid = c * NS + s
        r0 = s * rpt
        nzc = rpt // CH
        crow = wid * (epw // CH)

        pltpu.sync_copy(z_hbm, rows[0])

        def zbody(j, carry):
            pltpu.sync_copy(rows[0], h_sh.at[pl.ds(r0 + j * CH, CH)])
            return carry

        lax.fori_loop(0, nzc, zbody, 0)
        pltpu.sync_copy(src2_hbm.at[pl.ds(crow, NB)], srcb_v.at[pl.ds(0, NB)])
        pltpu.sync_copy(dst2_hbm.at[pl.ds(crow, NB)], dstb_v.at[pl.ds(0, NB)])
        plsc.subcore_barrier()

        def body(j, carry):
            par = lax.rem(j, 2)
            npar = 1 - par
            pbase = par * NB
            nxt = lax.rem(j + 1, nsc)
            di0 = pltpu.async_copy(
                src2_hbm.at[pl.ds(crow + nxt * NB, NB)],
                srcb_v.at[pl.ds(npar * NB, NB)], si0)
            di1 = pltpu.async_copy(
                dst2_hbm.at[pl.ds(crow + nxt * NB, NB)],
                dstb_v.at[pl.ds(npar * NB, NB)], si1)
            dg = [None] * DEPTH
            dsc = [None] * DEPTH
            for t in range(DEPTH - 1):
                dg[t] = pltpu.async_copy(
                    x_hbm.at[srcb_v.at[pbase + t]], rows[t], sem_r[t])
            for jj in range(NB):
                b = jj % DEPTH
                dg[b].wait()
                dsc[b] = pltpu.async_copy(
                    rows[b], h_sh.at[dstb_v.at[pbase + jj]],
                    sem_s[b], add=True)
                nxt_c = jj + DEPTH - 1
                if nxt_c < NB:
                    nb_ = nxt_c % DEPTH
                    if dsc[nb_] is not None:
                        dsc[nb_].wait()
                    dg[nb_] = pltpu.async_copy(
                        x_hbm.at[srcb_v.at[pbase + nxt_c]],
                        rows[nb_], sem_r[nb_])
            for jj in range(max(0, NB - DEPTH), NB):
                dsc[jj % DEPTH].wait()
            di0.wait()
            di1.wait()
            return carry

        lax.fori_loop(0, nsc, body, 0)
        plsc.subcore_barrier()

        def obody(j, carry):
            pltpu.sync_copy(h_sh.at[pl.ds(r0 + j * CH, CH)], rows[0])
            pltpu.sync_copy(
                rows[0], out_hbm.at[pl.ds(c * np_rows + r0 + j * CH, CH)])
            return carry

        lax.fori_loop(0, nzc, obody, 0)

    return segsum


def _deg_kernel(np_rows, rpt, epw, nch):
    mesh = plsc.VectorSubcoreMesh(core_axis_name="c", subcore_axis_name="s")

    @functools.partial(
        pl.kernel,
        out_type=jax.ShapeDtypeStruct((NC * np_rows, DW), jnp.float32),
        mesh=mesh,
        scratch_types=[
            pltpu.VMEM((1, CH), jnp.int32),
            pltpu.VMEM((CH, DW), jnp.float32),
            pltpu.VMEM_SHARED((np_rows, DW), jnp.float32),
        ],
    )
    def degk(dst2_hbm, ones_hbm, zd_hbm, out_hbm, dst_v, buf_v, d_sh):
        c = lax.axis_index("c")
        s = lax.axis_index("s")
        wid = c * NS + s
        r0 = s * rpt
        nzc = rpt // CH
        crow = wid * (epw // CH)
        pltpu.sync_copy(zd_hbm, buf_v)

        def zbody(j, carry):
            pltpu.sync_copy(buf_v, d_sh.at[pl.ds(r0 + j * CH, CH)])
            return carry

        lax.fori_loop(0, nzc, zbody, 0)
        pltpu.sync_copy(ones_hbm, buf_v)
        plsc.subcore_barrier()

        def body(g, carry):
            pltpu.sync_copy(dst2_hbm.at[pl.ds(crow + g, 1)], dst_v)
            pltpu.sync_copy(buf_v, d_sh.at[dst_v.at[0]], add=True)
            return carry

        lax.fori_loop(0, nch, body, 0)
        plsc.subcore_barrier()

        def obody(j, carry):
            pltpu.sync_copy(d_sh.at[pl.ds(r0 + j * CH, CH)], buf_v)
            pltpu.sync_copy(
                buf_v, out_hbm.at[pl.ds(c * np_rows + r0 + j * CH, CH)])
            return carry

        lax.fori_loop(0, nzc, obody, 0)

    return degk


def _mfconv_body(x_ref, hp_ref, dp_ref, wb_ref, xo_ref):
    h = hp_ref[0] + hp_ref[1]
    x = x_ref[...]
    z = jnp.dot(h, wb_ref[0:F], preferred_element_type=jnp.float32)
    z = z + jnp.dot(x, wb_ref[F:2 * F], preferred_element_type=jnp.float32)
    z = z + jnp.broadcast_to(wb_ref[2 * F:2 * F + 1], (BN, NK * F))
    d16 = jnp.minimum(dp_ref[0] + dp_ref[1], MAXD)
    deg = jnp.max(d16, axis=1, keepdims=True)
    kid = (lax.broadcasted_iota(jnp.int32, (BN, NK * F), 1) // F
           ).astype(jnp.float32)
    zm = jnp.where(kid == jnp.broadcast_to(deg, (BN, NK * F)), z, 0.0)
    acc = zm[:, 0:F]
    for k in range(1, NK):
        acc = acc + zm[:, k * F:(k + 1) * F]
    xo_ref[...] = jax.nn.sigmoid(acc)


def _mfconv_call(n_nodes, np_rows, x, hp, dp, wb):
    return pl.pallas_call(
        _mfconv_body,
        grid=(n_nodes // BN,),
        in_specs=[
            pl.BlockSpec((BN, F), lambda i: (i, 0)),
            pl.BlockSpec((NC, BN, F), lambda i: (0, i, 0)),
            pl.BlockSpec((NC, BN, DW), lambda i: (0, i, 0)),
            pl.BlockSpec((2 * F + 1, NK * F), lambda i: (0, 0)),
        ],
        out_specs=pl.BlockSpec((BN, F), lambda i: (i, 0)),
        out_shape=jax.ShapeDtypeStruct((np_rows, F), jnp.float32),
    )(x, hp, dp, wb)


def _finger_body(x_ref, wl_ref, fp_ref):
    i = pl.program_id(0)
    lg = jnp.dot(x_ref[...], wl_ref[0:F], preferred_element_type=jnp.float32)
    lg = lg + jnp.broadcast_to(wl_ref[F:F + 1], (BN, FPL))
    m = jnp.max(lg, axis=1, keepdims=True)
    e = jnp.exp(lg - m)
    y = e / jnp.sum(e, axis=1, keepdims=True)
    rowsum = jnp.sum(y, axis=0, keepdims=True)
    upd = jnp.broadcast_to(rowsum, (8, FPL))

    @pl.when(i == 0)
    def _():
        fp_ref[...] = upd

    @pl.when(i > 0)
    def _():
        fp_ref[...] = fp_ref[...] + upd


def _finger_call(n_nodes, x, wl):
    return pl.pallas_call(
        _finger_body,
        grid=(n_nodes // BN,),
        in_specs=[
            pl.BlockSpec((BN, F), lambda i: (i, 0)),
            pl.BlockSpec((F + 1, FPL), lambda i: (0, 0)),
        ],
        out_specs=pl.BlockSpec((8, FPL), lambda i: (0, 0)),
        out_shape=jax.ShapeDtypeStruct((8, FPL), jnp.float32),
    )(x, wl)


def kernel(x, edge_index, Wl, bl, Wr, Wlin, blin):
    n = x.shape[0]
    e = edge_index.shape[1]
    rpt = -(-(n + 1) // (NS * CH)) * CH
    np_rows = NS * rpt
    ep = -(-e // (NW * CH * NB)) * (NW * CH * NB)
    epw = ep // NW
    nch = epw // CH
    nsc = nch // NB

    src2 = jnp.concatenate(
        [edge_index[0].astype(jnp.int32),
         jnp.full((ep - e,), n, jnp.int32)]).reshape(ep // CH, CH)
    dst2 = jnp.concatenate(
        [edge_index[1].astype(jnp.int32),
         jnp.full((ep - e,), n, jnp.int32)]).reshape(ep // CH, CH)

    zrows = jnp.zeros((CH, F), jnp.float32)
    zd = jnp.zeros((CH, DW), jnp.float32)
    ones = jnp.ones((CH, DW), jnp.float32)

    segsum = _segsum_kernel(np_rows, rpt, epw, nsc)
    degk = _deg_kernel(np_rows, rpt, epw, nch)

    dp = degk(dst2, ones, zd).reshape(NC, np_rows, DW)

    w1 = jnp.transpose(Wl, (0, 3, 1, 2)).reshape(NL, F, NK * F)
    w2 = jnp.transpose(Wr, (0, 3, 1, 2)).reshape(NL, F, NK * F)
    wbig = jnp.concatenate([w1, w2, bl.reshape(NL, 1, NK * F)], axis=1)
    wlinb = jnp.concatenate(
        [jnp.transpose(Wlin, (0, 2, 1)), blin[:, None, :]], axis=1)

    xc = jnp.concatenate(
        [x.astype(jnp.float32), jnp.zeros((np_rows - n, F), jnp.float32)],
        axis=0)
    fps = []
    for layer in range(NL):
        hp = segsum(xc, src2, dst2, zrows).reshape(NC, np_rows, F)
        xc = _mfconv_call(n, np_rows, xc, hp, dp, wbig[layer])
        fps.append(_finger_call(n, xc, wlinb[layer]))
    return fps[0][0] + fps[1][0] + fps[2][0]


# speedup vs baseline: 1.1987x; 1.1987x over previous
import functools

import jax
import jax.numpy as jnp
from jax import lax
from jax.experimental import pallas as pl
from jax.experimental.pallas import tpu as pltpu
from jax.experimental.pallas import tpu_sc as plsc

F = 128
FPL = 2048
NK = 11
NL = 3
MAXD = 10.0

NC = 2
NS = 16
NW = NC * NS
CH = 128
DEPTH = 2
DW = 128

BN = 400


NB = 8


def _segsum_kernel(np_rows, rpt, a_sc, b_sc):
    mesh = plsc.VectorSubcoreMesh(core_axis_name="c", subcore_axis_name="s")

    @functools.partial(
        pl.kernel,
        out_type=jax.ShapeDtypeStruct((NC * np_rows, F), jnp.float32),
        mesh=mesh,
        scratch_types=[
            pltpu.VMEM((2 * NB, CH), jnp.int32),
            pltpu.VMEM((2 * NB, CH), jnp.int32),
            [pltpu.VMEM((CH, F), jnp.float32)] * DEPTH,
            pltpu.VMEM_SHARED((np_rows, F), jnp.float32),
            pltpu.SemaphoreType.DMA,
            pltpu.SemaphoreType.DMA,
            [pltpu.SemaphoreType.DMA] * DEPTH,
            [pltpu.SemaphoreType.DMA] * DEPTH,
        ],
    )
    def segsum(x_hbm, src2_hbm, dst2_hbm, z_hbm, out_hbm,
               srcb_v, dstb_v, rows, h_sh, si0, si1, sem_r, sem_s):
        c = lax.axis_index("c")
        s = lax.axis_index("s")
        r0 = s * rpt
        nzc = rpt // CH
        nsc = jnp.where(c == 0, a_sc, b_sc)
        crow = jnp.where(c == 0, s * (a_sc * NB),
                         NS * (a_sc * NB) + s * (b_sc * NB))

        pltpu.sync_copy(z_hbm, rows[0])

        def zbody(j, carry):
            pltpu.sync_copy(rows[0], h_sh.at[pl.ds(r0 + j * CH, CH)])
            return carry

        lax.fori_loop(0, nzc, zbody, 0)
        pltpu.sync_copy(src2_hbm.at[pl.ds(crow, NB)], srcb_v.at[pl.ds(0, NB)])
        pltpu.sync_copy(dst2_hbm.at[pl.ds(crow, NB)], dstb_v.at[pl.ds(0, NB)])
        plsc.subcore_barrier()

        def body(j, carry):
            par = lax.rem(j, 2)
            npar = 1 - par
            pbase = par * NB
            nxt = lax.rem(j + 1, nsc)
            di0 = pltpu.async_copy(
                src2_hbm.at[pl.ds(crow + nxt * NB, NB)],
                srcb_v.at[pl.ds(npar * NB, NB)], si0)
            di1 = pltpu.async_copy(
                dst2_hbm.at[pl.ds(crow + nxt * NB, NB)],
                dstb_v.at[pl.ds(npar * NB, NB)], si1)
            dg = [None] * DEPTH
            dsc = [None] * DEPTH
            for t in range(DEPTH - 1):
                dg[t] = pltpu.async_copy(
                    x_hbm.at[srcb_v.at[pbase + t]], rows[t], sem_r[t])
            for jj in range(NB):
                b = jj % DEPTH
                dg[b].wait()
                dsc[b] = pltpu.async_copy(
                    rows[b], h_sh.at[dstb_v.at[pbase + jj]],
                    sem_s[b], add=True)
                nxt_c = jj + DEPTH - 1
                if nxt_c < NB:
                    nb_ = nxt_c % DEPTH
                    if dsc[nb_] is not None:
                        dsc[nb_].wait()
                    dg[nb_] = pltpu.async_copy(
                        x_hbm.at[srcb_v.at[pbase + nxt_c]],
                        rows[nb_], sem_r[nb_])
            for jj in range(max(0, NB - DEPTH), NB):
                dsc[jj % DEPTH].wait()
            di0.wait()
            di1.wait()
            return carry

        lax.fori_loop(0, nsc, body, 0)
        plsc.subcore_barrier()

        def obody(j, carry):
            pltpu.sync_copy(h_sh.at[pl.ds(r0 + j * CH, CH)], rows[0])
            pltpu.sync_copy(
                rows[0], out_hbm.at[pl.ds(c * np_rows + r0 + j * CH, CH)])
            return carry

        lax.fori_loop(0, nzc, obody, 0)

    return segsum


def _deg_kernel(np_rows, rpt, epw, nch):
    mesh = plsc.VectorSubcoreMesh(core_axis_name="c", subcore_axis_name="s")

    @functools.partial(
        pl.kernel,
        out_type=jax.ShapeDtypeStruct((NC * np_rows, DW), jnp.float32),
        mesh=mesh,
        scratch_types=[
            pltpu.VMEM((1, CH), jnp.int32),
            pltpu.VMEM((CH, DW), jnp.float32),
            pltpu.VMEM_SHARED((np_rows, DW), jnp.float32),
        ],
    )
    def degk(dst2_hbm, ones_hbm, zd_hbm, out_hbm, dst_v, buf_v, d_sh):
        c = lax.axis_index("c")
        s = lax.axis_index("s")
        wid = c * NS + s
        r0 = s * rpt
        nzc = rpt // CH
        crow = wid * (epw // CH)
        pltpu.sync_copy(zd_hbm, buf_v)

        def zbody(j, carry):
            pltpu.sync_copy(buf_v, d_sh.at[pl.ds(r0 + j * CH, CH)])
            return carry

        lax.fori_loop(0, nzc, zbody, 0)
        pltpu.sync_copy(ones_hbm, buf_v)
        plsc.subcore_barrier()

        def body(g, carry):
            pltpu.sync_copy(dst2_hbm.at[pl.ds(crow + g, 1)], dst_v)
            pltpu.sync_copy(buf_v, d_sh.at[dst_v.at[0]], add=True)
            return carry

        lax.fori_loop(0, nch, body, 0)
        plsc.subcore_barrier()

        def obody(j, carry):
            pltpu.sync_copy(d_sh.at[pl.ds(r0 + j * CH, CH)], buf_v)
            pltpu.sync_copy(
                buf_v, out_hbm.at[pl.ds(c * np_rows + r0 + j * CH, CH)])
            return carry

        lax.fori_loop(0, nzc, obody, 0)

    return degk


def _mfconv_body(x_ref, hp_ref, dp_ref, wb_ref, xo_ref):
    h = hp_ref[0] + hp_ref[1]
    x = x_ref[...]
    z = jnp.dot(h, wb_ref[0:F], preferred_element_type=jnp.float32)
    z = z + jnp.dot(x, wb_ref[F:2 * F], preferred_element_type=jnp.float32)
    z = z + jnp.broadcast_to(wb_ref[2 * F:2 * F + 1], (BN, NK * F))
    d16 = jnp.minimum(dp_ref[0] + dp_ref[1], MAXD)
    deg = jnp.max(d16, axis=1, keepdims=True)
    kid = (lax.broadcasted_iota(jnp.int32, (BN, NK * F), 1) // F
           ).astype(jnp.float32)
    zm = jnp.where(kid == jnp.broadcast_to(deg, (BN, NK * F)), z, 0.0)
    acc = zm[:, 0:F]
    for k in range(1, NK):
        acc = acc + zm[:, k * F:(k + 1) * F]
    xo_ref[...] = jax.nn.sigmoid(acc)


def _mfconv_call(n_nodes, np_rows, x, hp, dp, wb):
    return pl.pallas_call(
        _mfconv_body,
        grid=(n_nodes // BN,),
        in_specs=[
            pl.BlockSpec((BN, F), lambda i: (i, 0)),
            pl.BlockSpec((NC, BN, F), lambda i: (0, i, 0)),
            pl.BlockSpec((NC, BN, DW), lambda i: (0, i, 0)),
            pl.BlockSpec((2 * F + 1, NK * F), lambda i: (0, 0)),
        ],
        out_specs=pl.BlockSpec((BN, F), lambda i: (i, 0)),
        out_shape=jax.ShapeDtypeStruct((np_rows, F), jnp.float32),
    )(x, hp, dp, wb)


def _finger_body(x_ref, wl_ref, fp_ref):
    i = pl.program_id(0)
    lg = jnp.dot(x_ref[...], wl_ref[0:F], preferred_element_type=jnp.float32)
    lg = lg + jnp.broadcast_to(wl_ref[F:F + 1], (BN, FPL))
    m = jnp.max(lg, axis=1, keepdims=True)
    e = jnp.exp(lg - m)
    y = e / jnp.sum(e, axis=1, keepdims=True)
    rowsum = jnp.sum(y, axis=0, keepdims=True)
    upd = jnp.broadcast_to(rowsum, (8, FPL))

    @pl.when(i == 0)
    def _():
        fp_ref[...] = upd

    @pl.when(i > 0)
    def _():
        fp_ref[...] = fp_ref[...] + upd


def _finger_call(n_nodes, x, wl):
    return pl.pallas_call(
        _finger_body,
        grid=(n_nodes // BN,),
        in_specs=[
            pl.BlockSpec((BN, F), lambda i: (i, 0)),
            pl.BlockSpec((F + 1, FPL), lambda i: (0, 0)),
        ],
        out_specs=pl.BlockSpec((8, FPL), lambda i: (0, 0)),
        out_shape=jax.ShapeDtypeStruct((8, FPL), jnp.float32),
    )(x, wl)


def kernel(x, edge_index, Wl, bl, Wr, Wlin, blin):
    n = x.shape[0]
    e = edge_index.shape[1]
    rpt = -(-(n + 1) // (NS * CH)) * CH
    np_rows = NS * rpt
    ep = -(-e // (NW * CH * NB)) * (NW * CH * NB)
    epw = ep // NW
    nch = epw // CH
    tot_sc = ep // (NS * CH * NB)
    a_sc = (3 * tot_sc) // 4
    b_sc = tot_sc - a_sc

    src2 = jnp.concatenate(
        [edge_index[0].astype(jnp.int32),
         jnp.full((ep - e,), n, jnp.int32)]).reshape(ep // CH, CH)
    dst2 = jnp.concatenate(
        [edge_index[1].astype(jnp.int32),
         jnp.full((ep - e,), n, jnp.int32)]).reshape(ep // CH, CH)

    zrows = jnp.zeros((CH, F), jnp.float32)
    zd = jnp.zeros((CH, DW), jnp.float32)
    ones = jnp.ones((CH, DW), jnp.float32)

    segsum = _segsum_kernel(np_rows, rpt, a_sc, b_sc)
    degk = _deg_kernel(np_rows, rpt, epw, nch)

    dp = degk(dst2, ones, zd).reshape(NC, np_rows, DW)

    w1 = jnp.transpose(Wl, (0, 3, 1, 2)).reshape(NL, F, NK * F)
    w2 = jnp.transpose(Wr, (0, 3, 1, 2)).reshape(NL, F, NK * F)
    wbig = jnp.concatenate([w1, w2, bl.reshape(NL, 1, NK * F)], axis=1)
    wlinb = jnp.concatenate(
        [jnp.transpose(Wlin, (0, 2, 1)), blin[:, None, :]], axis=1)

    xc = jnp.concatenate(
        [x.astype(jnp.float32), jnp.zeros((np_rows - n, F), jnp.float32)],
        axis=0)
    fps = []
    for layer in range(NL):
        hp = segsum(xc, src2, dst2, zrows).reshape(NC, np_rows, F)
        xc = _mfconv_call(n, np_rows, xc, hp, dp, wbig[layer])
        fps.append(_finger_call(n, xc, wlinb[layer]))
    return fps[0][0] + fps[1][0] + fps[2][0]
